# baseline (speedup 1.0000x reference)
import jax
import jax.numpy as jnp
from jax.experimental import pallas as pl
from jax.experimental.pallas import tpu as pltpu

_N_EMB = 8192
_DIM = 64
_BLK = 256
_HALF = _N_EMB // 2


def _vq_block(x_ref, w_ref, q_ref, enc_ref, idx_ref, wsq_ref):
    w = w_ref[...]

    @pl.when(pl.program_id(0) == 0)
    def _():
        wsq_ref[...] = jnp.sum(jnp.square(w), axis=1, keepdims=True).T

    x = x_ref[...]
    xsq = jnp.sum(jnp.square(x), axis=1, keepdims=True)
    wsq = wsq_ref[0, :]
    ndot2 = jax.lax.dot_general(
        x * (-2.0), w, dimension_numbers=(((1,), (1,)), ((), ())),
        preferred_element_type=jnp.float32)
    dist = (xsq + wsq) + ndot2

    d1 = dist[:, :_HALF]
    d2 = dist[:, _HALF:]
    hcol = jax.lax.broadcasted_iota(jnp.int32, d1.shape, 1)
    m1 = jnp.min(d1, axis=1, keepdims=True)
    i1 = jnp.min(jnp.where(d1 == m1, hcol, _N_EMB), axis=1)
    m2 = jnp.min(d2, axis=1, keepdims=True)
    i2 = jnp.min(jnp.where(d2 == m2, hcol, _N_EMB), axis=1) + _HALF
    m1b = m1.astype(jnp.bfloat16).astype(jnp.float32)
    idx = jnp.where(m2[:, 0] < m1b[:, 0], i2, i1)

    col = jax.lax.broadcasted_iota(jnp.int32, dist.shape, 1)
    enc = (col == idx[:, None]).astype(jnp.float32)
    enc_ref[...] = enc
    q_ref[...] = jax.lax.dot_general(
        enc, w, dimension_numbers=(((1,), (0,)), ((), ())),
        preferred_element_type=jnp.float32)
    idx_ref[...] = idx.reshape(1, 1, -1)


@jax.jit
def kernel(x, W):
    n_tokens = x.shape[0]
    n_blocks = n_tokens // _BLK
    quantized, encodings, idx3 = pl.pallas_call(
        _vq_block,
        grid=(n_blocks,),
        in_specs=[
            pl.BlockSpec((_BLK, _DIM), lambda i: (i, 0)),
            pl.BlockSpec((_N_EMB, _DIM), lambda i: (0, 0)),
        ],
        out_specs=[
            pl.BlockSpec((_BLK, _DIM), lambda i: (i, 0)),
            pl.BlockSpec((_BLK, _N_EMB), lambda i: (i, 0)),
            pl.BlockSpec((1, 1, _BLK), lambda i: (i, 0, 0)),
        ],
        out_shape=[
            jax.ShapeDtypeStruct((n_tokens, _DIM), jnp.float32),
            jax.ShapeDtypeStruct((n_tokens, _N_EMB), jnp.float32),
            jax.ShapeDtypeStruct((n_blocks, 1, _BLK), jnp.int32),
        ],
        scratch_shapes=[pltpu.VMEM((1, _N_EMB), jnp.float32)],
    )(x, W)
    return quantized, encodings, idx3.reshape(-1)


# speedup vs baseline: 1.1524x; 1.1524x over previous
import jax
import jax.numpy as jnp
from jax.experimental import pallas as pl

_N_EMB = 8192
_DIM = 64
_BLK = 512
_HALF = _N_EMB // 2


def _vq_block(x_ref, w_ref, q_ref, enc_ref, idx_ref):
    x = x_ref[...]
    w = w_ref[...]
    xsq = jnp.sum(jnp.square(x), axis=1, keepdims=True)
    wsq = jnp.sum(jnp.square(w), axis=1)
    dot = jax.lax.dot_general(
        x, w, dimension_numbers=(((1,), (1,)), ((), ())),
        preferred_element_type=jnp.float32)
    dist = xsq + wsq - 2.0 * dot

    d1 = dist[:, :_HALF]
    d2 = dist[:, _HALF:]
    hcol = jax.lax.broadcasted_iota(jnp.int32, d1.shape, 1)
    m1 = jnp.min(d1, axis=1, keepdims=True)
    i1 = jnp.min(jnp.where(d1 == m1, hcol, _N_EMB), axis=1)
    m2 = jnp.min(d2, axis=1, keepdims=True)
    i2 = jnp.min(jnp.where(d2 == m2, hcol, _N_EMB), axis=1) + _HALF
    m1b = m1.astype(jnp.bfloat16).astype(jnp.float32)
    idx = jnp.where(m2[:, 0] < m1b[:, 0], i2, i1)

    col = jax.lax.broadcasted_iota(jnp.int32, dist.shape, 1)
    enc = (col == idx[:, None]).astype(jnp.float32)
    enc_ref[...] = enc
    q_ref[...] = jax.lax.dot_general(
        enc, w, dimension_numbers=(((1,), (0,)), ((), ())),
        preferred_element_type=jnp.float32)
    idx_ref[...] = idx.reshape(1, 1, -1)


@jax.jit
def kernel(x, W):
    n_tokens = x.shape[0]
    n_blocks = n_tokens // _BLK
    quantized, encodings, idx3 = pl.pallas_call(
        _vq_block,
        grid=(n_blocks,),
        in_specs=[
            pl.BlockSpec((_BLK, _DIM), lambda i: (i, 0)),
            pl.BlockSpec((_N_EMB, _DIM), lambda i: (0, 0)),
        ],
        out_specs=[
            pl.BlockSpec((_BLK, _DIM), lambda i: (i, 0)),
            pl.BlockSpec((_BLK, _N_EMB), lambda i: (i, 0)),
            pl.BlockSpec((1, 1, _BLK), lambda i: (i, 0, 0)),
        ],
        out_shape=[
            jax.ShapeDtypeStruct((n_tokens, _DIM), jnp.float32),
            jax.ShapeDtypeStruct((n_tokens, _N_EMB), jnp.float32),
            jax.ShapeDtypeStruct((n_blocks, 1, _BLK), jnp.int32),
        ],
    )(x, W)
    return quantized, encodings, idx3.reshape(-1)
